# baseline (speedup 1.0000x reference)
import functools

import jax
import jax.numpy as jnp
from jax import lax
from jax.experimental import pallas as pl
from jax.experimental.pallas import tpu as pltpu
from jax.experimental.pallas import tpu_sc as plsc

NC, NS, LANES = 2, 16, 16
NW = NC * NS
G = 128
S = 256
NG = S // G
D_OUT = 64
DT = D_OUT // 8
TIR = 24
PITCH = 133


def _tc_tables(x_ref, p_ref, w_ref, adj_ref, a_ref, b_ref, i0_ref, i1_ref):
    f = jnp.dot(x_ref[...], p_ref[...], preferred_element_type=jnp.float32)
    f = f.astype(jnp.bfloat16)
    w = w_ref[...].astype(jnp.bfloat16)
    a_ref[...] = jnp.dot(f, w[:D_OUT, :], preferred_element_type=jnp.float32)
    b_ref[...] = jnp.dot(f, w[D_OUT:, :], preferred_element_type=jnp.float32)
    i0_ref[...] = adj_ref[0, :]
    i1_ref[...] = adj_ref[1, :]


def _sc_edge_body(nstages, a_hbm, b_hbm, i0_hbm, i1_hbm, out_hbm,
                  i0_v0, i0_v1, i1_v0, i1_v1, ra_v0, ra_v1, rb_v0, rb_v1,
                  ro_v0, ro_v1, sem_i0, sem_i1, sem_g0, sem_g1, sem_s0, sem_s1):
    i0_v, i1_v = (i0_v0, i0_v1), (i1_v0, i1_v1)
    ra_v, rb_v = (ra_v0, ra_v1), (rb_v0, rb_v1)
    ro_v = (ro_v0, ro_v1)
    sem_i, sem_g, sem_s = (sem_i0, sem_i1), (sem_g0, sem_g1), (sem_s0, sem_s1)

    wid = lax.axis_index("s") * NC + lax.axis_index("c")
    per = nstages // NW
    rem = nstages - per * NW
    nb = per + jnp.where(wid < rem, 1, 0)

    def issue_idx(j, s):
        blk = (wid + j * NW) * NG
        pltpu.async_copy(i0_hbm.at[pl.ds(blk, NG)], i0_v[s], sem_i[s])
        pltpu.async_copy(i1_hbm.at[pl.ds(blk, NG)], i1_v[s], sem_i[s])

    def wait_idx(s):
        pltpu.make_async_copy(i0_hbm.at[pl.ds(0, NG)], i0_v[s], sem_i[s]).wait()
        pltpu.make_async_copy(i1_hbm.at[pl.ds(0, NG)], i1_v[s], sem_i[s]).wait()

    def issue_gather(s):
        for h in range(NG):
            pltpu.async_copy(a_hbm.at[i0_v[s].at[h]],
                             ra_v[s].at[pl.ds(h * G, G)], sem_g[s])
            pltpu.async_copy(b_hbm.at[i1_v[s].at[h]],
                             rb_v[s].at[pl.ds(h * G, G)], sem_g[s])

    def wait_gather(s):
        pltpu.make_async_copy(a_hbm.at[pl.ds(0, S)], ra_v[s], sem_g[s]).wait()
        pltpu.make_async_copy(b_hbm.at[pl.ds(0, S)], rb_v[s], sem_g[s]).wait()

    lane = lax.iota(jnp.int32, LANES)
    row_base = [[(2 * c16 + (lane >> 3)) * TIR + (lane & 7) + 8 * tj
                 for c16 in range(D_OUT // LANES)] for tj in range(NG)]

    def compute(s):
        ra, rb, ro = ra_v[s], rb_v[s], ro_v[s]

        for tj in range(NG):
            @plsc.parallel_loop(tj * G, (tj + 1) * G, 1, unroll=4)
            def _(e, tj=tj):
                col = jnp.full((LANES,), e - tj * G, jnp.int32)
                for c32 in range(D_OUT // (2 * LANES)):
                    sl = pl.ds(c32 * 2 * LANES, 2 * LANES)
                    a0, a1 = plsc.unpack(ra[e, sl],
                                         format=plsc.PackFormat.INTERLEAVED)
                    b0, b1 = plsc.unpack(rb[e, sl],
                                         format=plsc.PackFormat.INTERLEAVED)
                    for k, v in ((0, a0 + b0), (1, a1 + b1)):
                        plsc.store_scatter(ro, [row_base[tj][2 * c32 + k], col],
                                           jnp.maximum(v, 0.2 * v))

    def issue_store(j, s):
        tjc = (wid + j * NW) * NG
        for ti in range(DT):
            pltpu.async_copy(
                ro_v[s].at[pl.ds(ti * TIR, NG * 8), pl.ds(0, G)],
                out_hbm.at[ti, pl.ds(tjc * 8, NG * 8)], sem_s[s])

    def wait_store(s):
        for ti in range(DT):
            pltpu.make_async_copy(
                ro_v[s].at[pl.ds(ti * TIR, NG * 8), pl.ds(0, G)],
                out_hbm.at[ti, pl.ds(0, NG * 8)], sem_s[s]).wait()

    issue_idx(0, 0)
    issue_idx(1, 1)
    wait_idx(0)
    issue_gather(0)

    def outer(jj, carry):
        for b in range(2):
            j = jj * 2 + b
            s, o = b, 1 - b

            @pl.when(j < nb)
            def _():
                @pl.when(j + 1 < nb)
                def _():
                    wait_idx(o)
                    issue_gather(o)

                wait_gather(s)

                @pl.when(j + 2 < nb)
                def _():
                    issue_idx(j + 2, s)

                @pl.when(j >= 2)
                def _():
                    wait_store(s)

                compute(s)
                issue_store(j, s)
        return carry

    lax.fori_loop(0, (nb + 1) // 2, outer, 0)
    wait_store(0)
    wait_store(1)


def kernel(disease_feats, gene_feats, chemical_feats, species_feats,
           trans_adj_list, pattern_name, P_disease, P_gene, P_chemical,
           P_species, W_DD):
    n, _ = disease_feats.shape
    e = trans_adj_list.shape[1]
    colperm = []
    for g in range(D_OUT // (2 * LANES)):
        for k in range(LANES):
            colperm += [g * 2 * LANES + k, g * 2 * LANES + LANES + k]
    w_perm = W_DD[:, jnp.array(colperm, jnp.int32)]
    a, b, i0, i1 = pl.pallas_call(
        _tc_tables,
        out_shape=[jax.ShapeDtypeStruct((n, D_OUT), jnp.float32)] * 2
        + [jax.ShapeDtypeStruct((e,), jnp.int32)] * 2,
    )(disease_feats, P_disease, w_perm, trans_adj_list)
    a = a.astype(jnp.bfloat16)
    b = b.astype(jnp.bfloat16)

    idx0 = i0.reshape(e // G, G)
    idx1 = i1.reshape(e // G, G)
    nstages = e // S
    ntc = e // G

    sc = pl.kernel(
        functools.partial(_sc_edge_body, nstages),
        out_type=jax.ShapeDtypeStruct((DT, ntc * 8, G), jnp.float32),
        mesh=plsc.VectorSubcoreMesh(core_axis_name="c", subcore_axis_name="s"),
        compiler_params=pltpu.CompilerParams(use_tc_tiling_on_sc=False,
                                             needs_layout_passes=False),
        scratch_types=[
            pltpu.VMEM((NG, G), jnp.int32),
            pltpu.VMEM((NG, G), jnp.int32),
            pltpu.VMEM((NG, G), jnp.int32),
            pltpu.VMEM((NG, G), jnp.int32),
            pltpu.VMEM((S, D_OUT), jnp.bfloat16),
            pltpu.VMEM((S, D_OUT), jnp.bfloat16),
            pltpu.VMEM((S, D_OUT), jnp.bfloat16),
            pltpu.VMEM((S, D_OUT), jnp.bfloat16),
            pltpu.VMEM((DT * TIR, PITCH), jnp.float32),
            pltpu.VMEM((DT * TIR, PITCH), jnp.float32),
            pltpu.SemaphoreType.DMA,
            pltpu.SemaphoreType.DMA,
            pltpu.SemaphoreType.DMA,
            pltpu.SemaphoreType.DMA,
            pltpu.SemaphoreType.DMA,
            pltpu.SemaphoreType.DMA,
        ],
    )
    out4 = sc(a, b, idx0, idx1).reshape(DT, ntc, 8, G)
    return jnp.transpose(out4, (1, 3, 0, 2)).reshape(e, D_OUT)


# speedup vs baseline: 1.0174x; 1.0174x over previous
import functools

import jax
import jax.numpy as jnp
from jax import lax
from jax.experimental import pallas as pl
from jax.experimental.pallas import tpu as pltpu
from jax.experimental.pallas import tpu_sc as plsc

NC, NS, LANES = 2, 16, 16
NW = NC * NS
G = 128
S = 256
NG = S // G
D_OUT = 64
DT = D_OUT // 8
TIR = 24
PITCH = 133


def _tc_tables(x_ref, p_ref, w_ref, adj_ref, a_ref, b_ref, i0_ref, i1_ref):
    f = jnp.dot(x_ref[...], p_ref[...], preferred_element_type=jnp.float32)
    f = f.astype(jnp.bfloat16)
    w = w_ref[...].astype(jnp.bfloat16)
    a = jnp.dot(f, w[:D_OUT, :], preferred_element_type=jnp.float32)
    b = jnp.dot(f, w[D_OUT:, :], preferred_element_type=jnp.float32)
    a_ref[...] = a.astype(jnp.bfloat16)
    b_ref[...] = b.astype(jnp.bfloat16)
    i0_ref[...] = adj_ref[0, :]
    i1_ref[...] = adj_ref[1, :]


def _sc_edge_body(nstages, a_hbm, b_hbm, i0_hbm, i1_hbm, out_hbm,
                  i0_v0, i0_v1, i1_v0, i1_v1, ra_v0, ra_v1, rb_v0, rb_v1,
                  ro_v0, ro_v1, sem_i0, sem_i1, sem_g0, sem_g1, sem_s0, sem_s1):
    i0_v, i1_v = (i0_v0, i0_v1), (i1_v0, i1_v1)
    ra_v, rb_v = (ra_v0, ra_v1), (rb_v0, rb_v1)
    ro_v = (ro_v0, ro_v1)
    sem_i, sem_g, sem_s = (sem_i0, sem_i1), (sem_g0, sem_g1), (sem_s0, sem_s1)

    wid = lax.axis_index("s") * NC + lax.axis_index("c")
    per = nstages // NW
    rem = nstages - per * NW
    nb = per + jnp.where(wid < rem, 1, 0)

    def issue_idx(j, s):
        blk = (wid + j * NW) * NG
        pltpu.async_copy(i0_hbm.at[pl.ds(blk, NG)], i0_v[s], sem_i[s])
        pltpu.async_copy(i1_hbm.at[pl.ds(blk, NG)], i1_v[s], sem_i[s])

    def wait_idx(s):
        pltpu.make_async_copy(i0_hbm.at[pl.ds(0, NG)], i0_v[s], sem_i[s]).wait()
        pltpu.make_async_copy(i1_hbm.at[pl.ds(0, NG)], i1_v[s], sem_i[s]).wait()

    def issue_gather(s):
        for h in range(NG):
            pltpu.async_copy(a_hbm.at[i0_v[s].at[h]],
                             ra_v[s].at[pl.ds(h * G, G)], sem_g[s])
            pltpu.async_copy(b_hbm.at[i1_v[s].at[h]],
                             rb_v[s].at[pl.ds(h * G, G)], sem_g[s])

    def wait_gather(s):
        pltpu.make_async_copy(a_hbm.at[pl.ds(0, S)], ra_v[s], sem_g[s]).wait()
        pltpu.make_async_copy(b_hbm.at[pl.ds(0, S)], rb_v[s], sem_g[s]).wait()

    lane = lax.iota(jnp.int32, LANES)
    row_base = [[(2 * c16 + (lane >> 3)) * TIR + (lane & 7) + 8 * tj
                 for c16 in range(D_OUT // LANES)] for tj in range(NG)]

    def compute(s):
        ra, rb, ro = ra_v[s], rb_v[s], ro_v[s]

        for tj in range(NG):
            @plsc.parallel_loop(tj * G, (tj + 1) * G, 1, unroll=4)
            def _(e, tj=tj):
                col = jnp.full((LANES,), e - tj * G, jnp.int32)
                for c32 in range(D_OUT // (2 * LANES)):
                    sl = pl.ds(c32 * 2 * LANES, 2 * LANES)
                    a0, a1 = plsc.unpack(ra[e, sl],
                                         format=plsc.PackFormat.INTERLEAVED)
                    b0, b1 = plsc.unpack(rb[e, sl],
                                         format=plsc.PackFormat.INTERLEAVED)
                    for k, v in ((0, a0 + b0), (1, a1 + b1)):
                        plsc.store_scatter(ro, [row_base[tj][2 * c32 + k], col],
                                           jnp.maximum(v, 0.2 * v))

    def issue_store(j, s):
        tjc = (wid + j * NW) * NG
        for ti in range(DT):
            pltpu.async_copy(
                ro_v[s].at[pl.ds(ti * TIR, NG * 8), pl.ds(0, G)],
                out_hbm.at[ti, pl.ds(tjc * 8, NG * 8)], sem_s[s])

    def wait_store(s):
        for ti in range(DT):
            pltpu.make_async_copy(
                ro_v[s].at[pl.ds(ti * TIR, NG * 8), pl.ds(0, G)],
                out_hbm.at[ti, pl.ds(0, NG * 8)], sem_s[s]).wait()

    issue_idx(0, 0)
    issue_idx(1, 1)
    wait_idx(0)
    issue_gather(0)

    def outer(jj, carry):
        for b in range(2):
            j = jj * 2 + b
            s, o = b, 1 - b

            @pl.when(j < nb)
            def _():
                @pl.when(j + 1 < nb)
                def _():
                    wait_idx(o)
                    issue_gather(o)

                wait_gather(s)

                @pl.when(j + 2 < nb)
                def _():
                    issue_idx(j + 2, s)

                @pl.when(j >= 2)
                def _():
                    wait_store(s)

                compute(s)
                issue_store(j, s)
        return carry

    lax.fori_loop(0, (nb + 1) // 2, outer, 0)
    wait_store(0)
    wait_store(1)


def kernel(disease_feats, gene_feats, chemical_feats, species_feats,
           trans_adj_list, pattern_name, P_disease, P_gene, P_chemical,
           P_species, W_DD):
    n, _ = disease_feats.shape
    e = trans_adj_list.shape[1]
    colperm = []
    for g in range(D_OUT // (2 * LANES)):
        for k in range(LANES):
            colperm += [g * 2 * LANES + k, g * 2 * LANES + LANES + k]
    w_perm = W_DD[:, jnp.array(colperm, jnp.int32)]
    a, b, i0, i1 = pl.pallas_call(
        _tc_tables,
        out_shape=[jax.ShapeDtypeStruct((n, D_OUT), jnp.bfloat16)] * 2
        + [jax.ShapeDtypeStruct((e,), jnp.int32)] * 2,
    )(disease_feats, P_disease, w_perm, trans_adj_list)

    idx0 = i0.reshape(e // G, G)
    idx1 = i1.reshape(e // G, G)
    nstages = e // S
    ntc = e // G

    sc = pl.kernel(
        functools.partial(_sc_edge_body, nstages),
        out_type=jax.ShapeDtypeStruct((DT, ntc * 8, G), jnp.float32),
        mesh=plsc.VectorSubcoreMesh(core_axis_name="c", subcore_axis_name="s"),
        compiler_params=pltpu.CompilerParams(use_tc_tiling_on_sc=False,
                                             needs_layout_passes=False),
        scratch_types=[
            pltpu.VMEM((NG, G), jnp.int32),
            pltpu.VMEM((NG, G), jnp.int32),
            pltpu.VMEM((NG, G), jnp.int32),
            pltpu.VMEM((NG, G), jnp.int32),
            pltpu.VMEM((S, D_OUT), jnp.bfloat16),
            pltpu.VMEM((S, D_OUT), jnp.bfloat16),
            pltpu.VMEM((S, D_OUT), jnp.bfloat16),
            pltpu.VMEM((S, D_OUT), jnp.bfloat16),
            pltpu.VMEM((DT * TIR, PITCH), jnp.float32),
            pltpu.VMEM((DT * TIR, PITCH), jnp.float32),
            pltpu.SemaphoreType.DMA,
            pltpu.SemaphoreType.DMA,
            pltpu.SemaphoreType.DMA,
            pltpu.SemaphoreType.DMA,
            pltpu.SemaphoreType.DMA,
            pltpu.SemaphoreType.DMA,
        ],
    )
    out4 = sc(a, b, idx0, idx1).reshape(DT, ntc, 8, G)
    return jnp.transpose(out4, (1, 3, 0, 2)).reshape(e, D_OUT)


# speedup vs baseline: 1.0235x; 1.0059x over previous
import functools

import jax
import jax.numpy as jnp
from jax import lax
from jax.experimental import pallas as pl
from jax.experimental.pallas import tpu as pltpu
from jax.experimental.pallas import tpu_sc as plsc

NC, NS, LANES = 2, 16, 16
NW = NC * NS
G = 128
S = 256
NG = S // G
D_OUT = 64
DT = D_OUT // 8
TIR = 24
PITCH = 133


def _tc_tables(x_ref, p_ref, w_ref, adj_ref, a_ref, b_ref, i0_ref, i1_ref):
    f = jnp.dot(x_ref[...], p_ref[...], preferred_element_type=jnp.float32)
    a = jnp.dot(f, w_ref[:D_OUT, :], preferred_element_type=jnp.float32)
    b = jnp.dot(f, w_ref[D_OUT:, :], preferred_element_type=jnp.float32)
    a_ref[...] = a.astype(jnp.bfloat16)
    b_ref[...] = b.astype(jnp.bfloat16)
    i0_ref[...] = adj_ref[0, :]
    i1_ref[...] = adj_ref[1, :]


def _sc_edge_body(nstages, a_hbm, b_hbm, i0_hbm, i1_hbm, out_hbm,
                  i0_v0, i0_v1, i1_v0, i1_v1, ra_v0, ra_v1, rb_v0, rb_v1,
                  ro_v0, ro_v1, sem_i0, sem_i1, sem_g0, sem_g1, sem_s0, sem_s1):
    i0_v, i1_v = (i0_v0, i0_v1), (i1_v0, i1_v1)
    ra_v, rb_v = (ra_v0, ra_v1), (rb_v0, rb_v1)
    ro_v = (ro_v0, ro_v1)
    sem_i, sem_g, sem_s = (sem_i0, sem_i1), (sem_g0, sem_g1), (sem_s0, sem_s1)

    wid = lax.axis_index("s") * NC + lax.axis_index("c")
    per = nstages // NW
    rem = nstages - per * NW
    nb = per + jnp.where(wid < rem, 1, 0)

    def issue_idx(j, s):
        blk = (wid + j * NW) * NG
        pltpu.async_copy(i0_hbm.at[pl.ds(blk, NG)], i0_v[s], sem_i[s])
        pltpu.async_copy(i1_hbm.at[pl.ds(blk, NG)], i1_v[s], sem_i[s])

    def wait_idx(s):
        pltpu.make_async_copy(i0_hbm.at[pl.ds(0, NG)], i0_v[s], sem_i[s]).wait()
        pltpu.make_async_copy(i1_hbm.at[pl.ds(0, NG)], i1_v[s], sem_i[s]).wait()

    def issue_gather(s):
        for h in range(NG):
            pltpu.async_copy(a_hbm.at[i0_v[s].at[h]],
                             ra_v[s].at[pl.ds(h * G, G)], sem_g[s])
            pltpu.async_copy(b_hbm.at[i1_v[s].at[h]],
                             rb_v[s].at[pl.ds(h * G, G)], sem_g[s])

    def wait_gather(s):
        pltpu.make_async_copy(a_hbm.at[pl.ds(0, S)], ra_v[s], sem_g[s]).wait()
        pltpu.make_async_copy(b_hbm.at[pl.ds(0, S)], rb_v[s], sem_g[s]).wait()

    lane = lax.iota(jnp.int32, LANES)
    row_base = [[(2 * c16 + (lane >> 3)) * TIR + (lane & 7) + 8 * tj
                 for c16 in range(D_OUT // LANES)] for tj in range(NG)]

    def compute(s):
        ra, rb, ro = ra_v[s], rb_v[s], ro_v[s]

        for tj in range(NG):
            @plsc.parallel_loop(tj * G, (tj + 1) * G, 1, unroll=4)
            def _(e, tj=tj):
                col = jnp.full((LANES,), e - tj * G, jnp.int32)
                for c32 in range(D_OUT // (2 * LANES)):
                    sl = pl.ds(c32 * 2 * LANES, 2 * LANES)
                    a0, a1 = plsc.unpack(ra[e, sl],
                                         format=plsc.PackFormat.INTERLEAVED)
                    b0, b1 = plsc.unpack(rb[e, sl],
                                         format=plsc.PackFormat.INTERLEAVED)
                    for k, v in ((0, a0 + b0), (1, a1 + b1)):
                        plsc.store_scatter(ro, [row_base[tj][2 * c32 + k], col],
                                           jnp.maximum(v, 0.2 * v))

    def issue_store(j, s):
        tjc = (wid + j * NW) * NG
        for ti in range(DT):
            pltpu.async_copy(
                ro_v[s].at[pl.ds(ti * TIR, NG * 8), pl.ds(0, G)],
                out_hbm.at[ti, pl.ds(tjc * 8, NG * 8)], sem_s[s])

    def wait_store(s):
        for ti in range(DT):
            pltpu.make_async_copy(
                ro_v[s].at[pl.ds(ti * TIR, NG * 8), pl.ds(0, G)],
                out_hbm.at[ti, pl.ds(0, NG * 8)], sem_s[s]).wait()

    issue_idx(0, 0)
    issue_idx(1, 1)
    wait_idx(0)
    issue_gather(0)

    def outer(jj, carry):
        for b in range(2):
            j = jj * 2 + b
            s, o = b, 1 - b

            @pl.when(j < nb)
            def _():
                @pl.when(j + 1 < nb)
                def _():
                    wait_idx(o)
                    issue_gather(o)

                wait_gather(s)

                @pl.when(j + 2 < nb)
                def _():
                    issue_idx(j + 2, s)

                @pl.when(j >= 2)
                def _():
                    wait_store(s)

                compute(s)
                issue_store(j, s)
        return carry

    lax.fori_loop(0, (nb + 1) // 2, outer, 0)
    wait_store(0)
    wait_store(1)


def kernel(disease_feats, gene_feats, chemical_feats, species_feats,
           trans_adj_list, pattern_name, P_disease, P_gene, P_chemical,
           P_species, W_DD):
    n, _ = disease_feats.shape
    e = trans_adj_list.shape[1]
    colperm = []
    for g in range(D_OUT // (2 * LANES)):
        for k in range(LANES):
            colperm += [g * 2 * LANES + k, g * 2 * LANES + LANES + k]
    w_perm = W_DD[:, jnp.array(colperm, jnp.int32)]
    a, b, i0, i1 = pl.pallas_call(
        _tc_tables,
        out_shape=[jax.ShapeDtypeStruct((n, D_OUT), jnp.bfloat16)] * 2
        + [jax.ShapeDtypeStruct((e,), jnp.int32)] * 2,
    )(disease_feats, P_disease, w_perm, trans_adj_list)

    idx0 = i0.reshape(e // G, G)
    idx1 = i1.reshape(e // G, G)
    nstages = e // S
    ntc = e // G

    sc = pl.kernel(
        functools.partial(_sc_edge_body, nstages),
        out_type=jax.ShapeDtypeStruct((DT, ntc * 8, G), jnp.float32),
        mesh=plsc.VectorSubcoreMesh(core_axis_name="c", subcore_axis_name="s"),
        compiler_params=pltpu.CompilerParams(use_tc_tiling_on_sc=False,
                                             needs_layout_passes=False),
        scratch_types=[
            pltpu.VMEM((NG, G), jnp.int32),
            pltpu.VMEM((NG, G), jnp.int32),
            pltpu.VMEM((NG, G), jnp.int32),
            pltpu.VMEM((NG, G), jnp.int32),
            pltpu.VMEM((S, D_OUT), jnp.bfloat16),
            pltpu.VMEM((S, D_OUT), jnp.bfloat16),
            pltpu.VMEM((S, D_OUT), jnp.bfloat16),
            pltpu.VMEM((S, D_OUT), jnp.bfloat16),
            pltpu.VMEM((DT * TIR, PITCH), jnp.float32),
            pltpu.VMEM((DT * TIR, PITCH), jnp.float32),
            pltpu.SemaphoreType.DMA,
            pltpu.SemaphoreType.DMA,
            pltpu.SemaphoreType.DMA,
            pltpu.SemaphoreType.DMA,
            pltpu.SemaphoreType.DMA,
            pltpu.SemaphoreType.DMA,
        ],
    )
    out4 = sc(a, b, idx0, idx1).reshape(DT, ntc, 8, G)
    return jnp.transpose(out4, (1, 3, 0, 2)).reshape(e, D_OUT)
